# Initial kernel scaffold; baseline (speedup 1.0000x reference)
import jax
import jax.numpy as jnp
from jax.experimental import pallas as pl


def kernel(x, token_table, pos_table):
    raise NotImplementedError("write your pallas kernel here")


# speedup vs baseline: 4.2459x; 4.2459x over previous
import functools

import jax
import jax.numpy as jnp
from jax import lax
from jax.experimental import pallas as pl
from jax.experimental.pallas import tpu as pltpu
from jax.experimental.pallas import tpu_sc as plsc

VOCAB = 100000
MAXLEN = 200
EMBED = 64
BATCH = 4096

NC, NS = 2, 16
NW = NC * NS
BPW = BATCH // NW
NBUF = 4
NSTEP = MAXLEN // NBUF


def _tpe_body(tok_hbm, xt_hbm, pos_hbm, out_hbm,
              idx_v, pos_v,
              g0, g1, g2, g3, o0, o1, o2, o3,
              gs0, gs1, gs2, gs3, os0, os1, os2, os3):
    gbufs = (g0, g1, g2, g3)
    obufs = (o0, o1, o2, o3)
    gsems = (gs0, gs1, gs2, gs3)
    osems = (os0, os1, os2, os3)

    wid = lax.axis_index("s") * NC + lax.axis_index("c")
    b0 = wid * BPW

    pltpu.sync_copy(xt_hbm.at[:, pl.ds(b0, BPW)], idx_v)
    pltpu.sync_copy(pos_hbm, pos_v)

    def fire_gather(s, j):
        pltpu.async_copy(tok_hbm.at[idx_v.at[j]], gbufs[s], gsems[s])

    def wait_gather(s):
        pltpu.make_async_copy(tok_hbm.at[idx_v.at[0]], gbufs[s], gsems[s]).wait()

    def fire_scatter(s, j):
        pltpu.async_copy(obufs[s], out_hbm.at[pl.ds(b0, BPW), pl.ds(j, 1)],
                         osems[s])

    def wait_scatter(s):
        pltpu.make_async_copy(obufs[s],
                              out_hbm.at[pl.ds(b0, BPW), pl.ds(0, 1)],
                              osems[s]).wait()

    def add_pos(s, j):
        p = [pos_v[j, pl.ds(16 * k, 16)] for k in range(4)]
        gb, ob = gbufs[s], obufs[s]

        def row_body(i, _):
            r = 4 * i
            for u in range(4):
                for k in range(4):
                    ob[r + u, 0, pl.ds(16 * k, 16)] = (
                        gb[r + u, pl.ds(16 * k, 16)] + p[k])
            return 0

        lax.fori_loop(0, BPW // 4, row_body, 0)

    for s in range(NBUF):
        fire_gather(s, s)

    def turn(t, _):
        for s in range(NBUF):
            j = NBUF * t + s
            wait_gather(s)

            @pl.when(t > 0)
            def _():
                wait_scatter(s)

            add_pos(s, j)
            fire_gather(s, jnp.minimum(j + NBUF, MAXLEN - 1))
            fire_scatter(s, j)
        return 0

    lax.fori_loop(0, NSTEP, turn, 0)

    for s in range(NBUF):
        wait_gather(s)
        wait_scatter(s)


_mesh = plsc.VectorSubcoreMesh(core_axis_name="c", subcore_axis_name="s")

_tpe_call = functools.partial(
    pl.kernel,
    out_type=jax.ShapeDtypeStruct((BATCH, MAXLEN, EMBED), jnp.float32),
    mesh=_mesh,
    scratch_types=[
        pltpu.VMEM((MAXLEN, BPW), jnp.int32),
        pltpu.VMEM((MAXLEN, EMBED), jnp.float32),
    ]
    + [pltpu.VMEM((BPW, EMBED), jnp.float32) for _ in range(NBUF)]
    + [pltpu.VMEM((BPW, 1, EMBED), jnp.float32) for _ in range(NBUF)]
    + [pltpu.SemaphoreType.DMA for _ in range(2 * NBUF)],
    compiler_params=pltpu.CompilerParams(use_tc_tiling_on_sc=False),
)(_tpe_body)


@jax.jit
def kernel(x, token_table, pos_table):
    xt = jnp.transpose(x.astype(jnp.int32))
    return _tpe_call(token_table, xt, pos_table)
